# baseline (speedup 1.0000x reference)
import functools

import jax
import jax.numpy as jnp
from jax import lax
from jax.experimental import pallas as pl
from jax.experimental.pallas import tpu as pltpu
from jax.experimental.pallas import tpu_sc as plsc

_GRP = 16
_LANE = 128
_NCHUNK = 1


def _sc_gather_cols(jobs):
    D = jobs[0][1].shape[0]
    info = plsc.get_sparse_core_info()
    NC, NS = info.num_cores, info.num_subcores
    NW = NC * NS
    bws = [idx.shape[0] // NW for idx, _ in jobs]

    mesh = plsc.VectorSubcoreMesh(core_axis_name="c", subcore_axis_name="s")

    @functools.partial(
        pl.kernel,
        out_type=tuple(
            jax.ShapeDtypeStruct((idx.shape[0], D), jnp.float32)
            for idx, _ in jobs),
        mesh=mesh,
        scratch_types=(
            [pltpu.VMEM((bw + _GRP,), jnp.int32) for bw in bws]
            + [pltpu.VMEM((_GRP, D, _LANE), jnp.float32)]
            + [pltpu.VMEM((bw, D), jnp.float32) for bw in bws]
            + [pltpu.SemaphoreType.DMA] * _GRP
        ),
        compiler_params=pltpu.CompilerParams(needs_layout_passes=False),
    )
    def gather_k(*refs):
        n = len(jobs)
        idx_hbms = refs[:n]
        tab_hbms = refs[n:2 * n]
        out_hbms = refs[2 * n:3 * n]
        idx_vs = refs[3 * n:4 * n]
        ring_v = refs[4 * n]
        out_vs = refs[4 * n + 1:5 * n + 1]
        sems = refs[5 * n + 1:5 * n + 1 + _GRP]

        wid = lax.axis_index("s") * NC + lax.axis_index("c")
        iota = lax.iota(jnp.int32, 16)
        d_lo = iota
        d_hi = iota + 16

        for j in range(n):
            base = pl.multiple_of(wid * bws[j], 8)
            pltpu.sync_copy(idx_hbms[j].at[pl.ds(base, bws[j])],
                            idx_vs[j].at[pl.ds(0, bws[j])])

        def slab(tab, idx16, k):
            cb = pl.multiple_of((idx16[k] >> 7) << 7, _LANE)
            return tab.at[:, pl.ds(cb, _LANE)]

        for j in range(n):
            tab, idx_v, out_v = tab_hbms[j], idx_vs[j], out_vs[j]
            n_grp = bws[j] // _GRP

            idx0 = idx_v[pl.ds(0, _GRP)]
            for k in range(_GRP):
                pltpu.async_copy(slab(tab, idx0, k), ring_v.at[k], sems[k])

            def group(gi, _, tab=tab, idx_v=idx_v, out_v=out_v,
                      n_grp=n_grp):
                cur = idx_v[pl.ds(gi * _GRP, _GRP)]
                nxt = idx_v[pl.ds((gi + 1) * _GRP, _GRP)]
                for k in range(_GRP):
                    pltpu.make_async_copy(tab.at[:, pl.ds(0, _LANE)],
                                          ring_v.at[k], sems[k]).wait()
                    o = cur[k] & (_LANE - 1)
                    slot_v = jnp.full((16,), k, jnp.int32)
                    col_v = jnp.full((16,), o, jnp.int32)
                    g_v = jnp.full((16,), gi * _GRP + k, jnp.int32)
                    lo = plsc.load_gather(ring_v, [slot_v, d_lo, col_v])
                    hi = plsc.load_gather(ring_v, [slot_v, d_hi, col_v])
                    plsc.store_scatter(out_v, [g_v, d_lo], lo)
                    plsc.store_scatter(out_v, [g_v, d_hi], hi)

                    @pl.when(gi + 1 < n_grp)
                    def _():
                        pltpu.async_copy(slab(tab, nxt, k),
                                         ring_v.at[k], sems[k])
                return _

            lax.fori_loop(0, n_grp, group, None)

        for j in range(n):
            base = pl.multiple_of(wid * bws[j], 8)
            pltpu.sync_copy(out_vs[j],
                            out_hbms[j].at[pl.ds(base, bws[j]), :])

    args = [idx for idx, _ in jobs] + [tab for _, tab in jobs]
    return gather_k(*args)


def _tc_matmul_chunk(big, p_c, q_i, c):
    B, D = q_i.shape
    L = p_c.shape[0]
    BM = 1024
    nb = L // BM

    def body(*refs):
        p_ref, q_ref, o_ref = refs[-3], refs[-2], refs[-1]
        o_ref[...] = lax.dot_general(
            p_ref[...], q_ref[...],
            dimension_numbers=(((1,), (1,)), ((), ())),
            preferred_element_type=jnp.float32,
        )

    in_specs = [
        pl.BlockSpec((BM, D), lambda i: (i, 0)),
        pl.BlockSpec((B, D), lambda i: (0, 0)),
    ]
    inputs = (p_c, q_i)
    kwargs = {}
    if big is not None:
        in_specs = [pl.BlockSpec(memory_space=pl.ANY)] + in_specs
        inputs = (big,) + inputs
        kwargs["input_output_aliases"] = {0: 0}

    def out_map(i, c=c, nb=nb):
        return (c * nb + i, 0)

    return pl.pallas_call(
        body,
        grid=(nb,),
        in_specs=in_specs,
        out_specs=pl.BlockSpec((BM, B), out_map),
        out_shape=jax.ShapeDtypeStruct((B, B), jnp.float32),
        **kwargs,
    )(*inputs)


def kernel(user_id, item_id, user_embed, item_embed):
    uid = user_id.astype(jnp.int32)
    iid = item_id.astype(jnp.int32)
    uemb_t = user_embed.T
    iemb_t = item_embed.T
    B = uid.shape[0]
    L = B // _NCHUNK

    qi, pu0 = _sc_gather_cols([(iid, iemb_t), (uid[:L], uemb_t)])
    pus = [pu0] + [
        _sc_gather_cols([(uid[c * L:(c + 1) * L], uemb_t)])[0]
        for c in range(1, _NCHUNK)
    ]

    big = None
    for c in range(_NCHUNK):
        big = _tc_matmul_chunk(big, pus[c], qi, c)
    return big


# speedup vs baseline: 1.0225x; 1.0225x over previous
import functools

import jax
import jax.numpy as jnp
from jax import lax
from jax.experimental import pallas as pl
from jax.experimental.pallas import tpu as pltpu
from jax.experimental.pallas import tpu_sc as plsc

_GRP = 16
_LANE = 128
_NCHUNK = 1


def _sc_gather_cols(jobs):
    D = jobs[0][1].shape[0]
    info = plsc.get_sparse_core_info()
    NC, NS = info.num_cores, info.num_subcores
    NW = NC * NS
    bws = [idx.shape[0] // NW for idx, _ in jobs]

    mesh = plsc.VectorSubcoreMesh(core_axis_name="c", subcore_axis_name="s")

    @functools.partial(
        pl.kernel,
        out_type=tuple(
            jax.ShapeDtypeStruct((idx.shape[0], D), jnp.float32)
            for idx, _ in jobs),
        mesh=mesh,
        scratch_types=(
            [pltpu.VMEM((bw + _GRP,), jnp.int32) for bw in bws]
            + [pltpu.VMEM((_GRP, D, _LANE), jnp.float32)]
            + [pltpu.VMEM((bw, D), jnp.float32) for bw in bws]
            + [pltpu.SemaphoreType.DMA] * _GRP
        ),
        compiler_params=pltpu.CompilerParams(needs_layout_passes=False),
    )
    def gather_k(*refs):
        n = len(jobs)
        idx_hbms = refs[:n]
        tab_hbms = refs[n:2 * n]
        out_hbms = refs[2 * n:3 * n]
        idx_vs = refs[3 * n:4 * n]
        ring_v = refs[4 * n]
        out_vs = refs[4 * n + 1:5 * n + 1]
        sems = refs[5 * n + 1:5 * n + 1 + _GRP]

        wid = lax.axis_index("s") * NC + lax.axis_index("c")
        iota = lax.iota(jnp.int32, 16)
        d_lo = iota
        d_hi = iota + 16

        for j in range(n):
            base = pl.multiple_of(wid * bws[j], 8)
            pltpu.sync_copy(idx_hbms[j].at[pl.ds(base, bws[j])],
                            idx_vs[j].at[pl.ds(0, bws[j])])

        def slab(tab, idx16, k):
            cb = pl.multiple_of((idx16[k] >> 7) << 7, _LANE)
            return tab.at[:, pl.ds(cb, _LANE)]

        for j in range(n):
            tab, idx_v, out_v = tab_hbms[j], idx_vs[j], out_vs[j]
            n_grp = bws[j] // _GRP

            idx0 = idx_v[pl.ds(0, _GRP)]
            for k in range(_GRP):
                pltpu.async_copy(slab(tab, idx0, k), ring_v.at[k], sems[k])

            def group(gi, _, tab=tab, idx_v=idx_v, out_v=out_v,
                      n_grp=n_grp):
                cur = idx_v[pl.ds(gi * _GRP, _GRP)]
                nxt = idx_v[pl.ds((gi + 1) * _GRP, _GRP)]
                for k in range(_GRP):
                    pltpu.make_async_copy(tab.at[:, pl.ds(0, _LANE)],
                                          ring_v.at[k], sems[k]).wait()
                    o = cur[k] & (_LANE - 1)
                    slot_v = jnp.full((16,), k, jnp.int32)
                    col_v = jnp.full((16,), o, jnp.int32)
                    g_v = jnp.full((16,), gi * _GRP + k, jnp.int32)
                    lo = plsc.load_gather(ring_v, [slot_v, d_lo, col_v])
                    hi = plsc.load_gather(ring_v, [slot_v, d_hi, col_v])
                    plsc.store_scatter(out_v, [g_v, d_lo], lo)
                    plsc.store_scatter(out_v, [g_v, d_hi], hi)

                    @pl.when(gi + 1 < n_grp)
                    def _():
                        pltpu.async_copy(slab(tab, nxt, k),
                                         ring_v.at[k], sems[k])
                return _

            lax.fori_loop(0, n_grp, group, None)

        for j in range(n):
            base = pl.multiple_of(wid * bws[j], 8)
            pltpu.sync_copy(out_vs[j],
                            out_hbms[j].at[pl.ds(base, bws[j]), :])

    args = [idx for idx, _ in jobs] + [tab for _, tab in jobs]
    return gather_k(*args)


def _tc_matmul_chunk(big, p_c, q_i, c):
    B, D = q_i.shape
    L = p_c.shape[0]
    BM = 512
    nb = L // BM

    def body(*refs):
        p_ref, q_ref, o_ref = refs[-3], refs[-2], refs[-1]
        o_ref[...] = lax.dot_general(
            p_ref[...], q_ref[...],
            dimension_numbers=(((1,), (1,)), ((), ())),
            preferred_element_type=jnp.float32,
        )

    in_specs = [
        pl.BlockSpec((BM, D), lambda i: (i, 0)),
        pl.BlockSpec((B, D), lambda i: (0, 0)),
    ]
    inputs = (p_c, q_i)
    kwargs = {}
    if big is not None:
        in_specs = [pl.BlockSpec(memory_space=pl.ANY)] + in_specs
        inputs = (big,) + inputs
        kwargs["input_output_aliases"] = {0: 0}

    def out_map(i, c=c, nb=nb):
        return (c * nb + i, 0)

    return pl.pallas_call(
        body,
        grid=(nb,),
        in_specs=in_specs,
        out_specs=pl.BlockSpec((BM, B), out_map),
        out_shape=jax.ShapeDtypeStruct((B, B), jnp.float32),
        **kwargs,
    )(*inputs)


def kernel(user_id, item_id, user_embed, item_embed):
    uid = user_id.astype(jnp.int32)
    iid = item_id.astype(jnp.int32)
    uemb_t = user_embed.T
    iemb_t = item_embed.T
    B = uid.shape[0]
    L = B // _NCHUNK

    qi, pu0 = _sc_gather_cols([(iid, iemb_t), (uid[:L], uemb_t)])
    pus = [pu0] + [
        _sc_gather_cols([(uid[c * L:(c + 1) * L], uemb_t)])[0]
        for c in range(1, _NCHUNK)
    ]

    big = None
    for c in range(_NCHUNK):
        big = _tc_matmul_chunk(big, pus[c], qi, c)
    return big
